# baseline (speedup 1.0000x reference)
import functools

import jax
import jax.numpy as jnp
from jax import lax
from jax.experimental import pallas as pl
from jax.experimental.pallas import tpu as pltpu
from jax.experimental.pallas import tpu_sc as plsc

_F0, _F1, _F2 = 7500, 2000, 500
_H = 512

_NC = 2
_NT = 16


def _leaky(v):
    return jnp.where(v > 0, v, 0.01 * v)


def _mm_leaky_body(x_ref, w_ref, b_ref, o_ref):
    y = jnp.dot(x_ref[...], w_ref[...], preferred_element_type=jnp.float32)
    o_ref[...] = _leaky(y + b_ref[...])


def _mm_leaky(x, w, b, block_r):
    n, k = x.shape
    m = w.shape[1]
    return pl.pallas_call(
        _mm_leaky_body,
        grid=(n // block_r,),
        in_specs=[
            pl.BlockSpec((block_r, k), lambda i: (i, 0)),
            pl.BlockSpec((k, m), lambda i: (0, 0)),
            pl.BlockSpec((1, m), lambda i: (0, 0)),
        ],
        out_specs=pl.BlockSpec((block_r, m), lambda i: (i, 0)),
        out_shape=jax.ShapeDtypeStruct((n, m), jnp.float32),
    )(x, w, b.reshape(1, m))


def _conv_post_body(n_chunks, block_r, s_ref, c_ref, w_ref, b_ref, g_ref, bt_ref, o_ref):
    s = s_ref[...]
    agg = jnp.concatenate([s[j] for j in range(n_chunks)], axis=-1)
    cp = c_ref[...]
    cnt = jnp.clip((cp[0, :, :1] + cp[1, :, :1]) * 0.5, 1.0, None)
    h = jnp.dot(agg / cnt, w_ref[...], preferred_element_type=jnp.float32)
    h = h + b_ref[...]
    mu = jnp.mean(h, axis=-1, keepdims=True)
    var = jnp.mean((h - mu) ** 2, axis=-1, keepdims=True)
    h = (h - mu) / jnp.sqrt(var + 1e-5) * g_ref[...] + bt_ref[...]
    o_ref[...] = _leaky(h)


def _conv_post(sums, cnt, w, b, g, bt, block_r):
    n_rows = sums.shape[1]
    n_chunks, _, cf = sums.shape
    m = w.shape[1]
    return pl.pallas_call(
        functools.partial(_conv_post_body, n_chunks, block_r),
        grid=(n_rows // block_r,),
        in_specs=[
            pl.BlockSpec((n_chunks, block_r, cf), lambda i: (0, i, 0)),
            pl.BlockSpec((2, block_r, cf), lambda i: (0, i, 0)),
            pl.BlockSpec((_H, m), lambda i: (0, 0)),
            pl.BlockSpec((1, m), lambda i: (0, 0)),
            pl.BlockSpec((1, m), lambda i: (0, 0)),
            pl.BlockSpec((1, m), lambda i: (0, 0)),
        ],
        out_specs=pl.BlockSpec((block_r, m), lambda i: (i, 0)),
        out_shape=jax.ShapeDtypeStruct((n_rows, m), jnp.float32),
    )(sums, cnt, w, b.reshape(1, m), g.reshape(1, m), bt.reshape(1, m))


def _att_a_body(x_ref, wl_ref, bl_ref, w1_ref, b1_ref, w2_ref, b2_ref,
                w3_ref, b3_ref, xs_ref, lg_ref):
    xs = _leaky(jnp.dot(x_ref[...], wl_ref[...],
                        preferred_element_type=jnp.float32) + bl_ref[...])
    a1 = jnp.tanh(jnp.dot(xs, w1_ref[...],
                          preferred_element_type=jnp.float32) + b1_ref[...])
    a2 = jax.nn.sigmoid(jnp.dot(xs, w2_ref[...],
                                preferred_element_type=jnp.float32) + b2_ref[...])
    lg = jnp.dot(a1 * a2, w3_ref[...],
                 preferred_element_type=jnp.float32) + b3_ref[...]
    xs_ref[...] = xs
    lg_ref[...] = lg


def _att_a(xin, wl, bl, w1, b1, w2, b2, w3, b3, block_r):
    n = xin.shape[0]
    return pl.pallas_call(
        _att_a_body,
        grid=(n // block_r,),
        in_specs=[
            pl.BlockSpec((block_r, _H), lambda i: (i, 0)),
            pl.BlockSpec((_H, _H), lambda i: (0, 0)),
            pl.BlockSpec((1, _H), lambda i: (0, 0)),
            pl.BlockSpec((_H, _H), lambda i: (0, 0)),
            pl.BlockSpec((1, _H), lambda i: (0, 0)),
            pl.BlockSpec((_H, _H), lambda i: (0, 0)),
            pl.BlockSpec((1, _H), lambda i: (0, 0)),
            pl.BlockSpec((_H, 1), lambda i: (0, 0)),
            pl.BlockSpec((1, 1), lambda i: (0, 0)),
        ],
        out_specs=[
            pl.BlockSpec((block_r, _H), lambda i: (i, 0)),
            pl.BlockSpec((block_r, 1), lambda i: (i, 0)),
        ],
        out_shape=[
            jax.ShapeDtypeStruct((n, _H), jnp.float32),
            jax.ShapeDtypeStruct((n, 1), jnp.float32),
        ],
    )(xin, wl, bl.reshape(1, _H), w1, b1.reshape(1, _H), w2, b2.reshape(1, _H),
      w3, b3.reshape(1, 1))


def _att_b_body(block_r, n_valid, lg_ref, xs_ref, a_ref, xv_ref):
    i = pl.program_id(0)
    lg = lg_ref[...]
    if n_valid != lg.shape[0]:
        mask = lax.broadcasted_iota(jnp.int32, lg.shape, 0) < n_valid
        lg = jnp.where(mask, lg, -1e30)
    m = jnp.max(lg)
    den = jnp.sum(jnp.exp(lg - m))
    lgb = lg_ref[pl.ds(i * block_r, block_r), :]
    if n_valid != lg.shape[0]:
        bmask = (lax.broadcasted_iota(jnp.int32, (block_r, 1), 0)
                 + i * block_r) < n_valid
        lgb = jnp.where(bmask, lgb, -1e30)
    eb = jnp.exp(lgb - m) / den
    a_ref[...] = eb
    contrib = lax.dot_general(eb, xs_ref[...], (((0,), (0,)), ((), ())),
                              preferred_element_type=jnp.float32)

    @pl.when(i == 0)
    def _():
        xv_ref[...] = jnp.zeros_like(xv_ref)

    xv_ref[...] += contrib


def _att_b(lg, xs, block_r, n_valid):
    n = xs.shape[0]
    return pl.pallas_call(
        functools.partial(_att_b_body, block_r, n_valid),
        grid=(n // block_r,),
        in_specs=[
            pl.BlockSpec((n, 1), lambda i: (0, 0)),
            pl.BlockSpec((block_r, _H), lambda i: (i, 0)),
        ],
        out_specs=[
            pl.BlockSpec((block_r, 1), lambda i: (i, 0)),
            pl.BlockSpec((1, _H), lambda i: (0, 0)),
        ],
        out_shape=[
            jax.ShapeDtypeStruct((n, 1), jnp.float32),
            jax.ShapeDtypeStruct((1, _H), jnp.float32),
        ],
    )(lg, xs)


def _head_body(xv_ref, w1_ref, b1_ref, w2_ref, b2_ref, o_ref):
    h = _leaky(jnp.dot(xv_ref[...], w1_ref[...],
                       preferred_element_type=jnp.float32) + b1_ref[...])
    o_ref[...] = jax.nn.sigmoid(
        jnp.dot(h, w2_ref[...], preferred_element_type=jnp.float32) + b2_ref[...])


def _head(xv, w1, b1, w2, b2):
    k = w1.shape[0]
    m = w2.shape[1]
    return pl.pallas_call(
        _head_body,
        in_specs=[
            pl.BlockSpec((1, k), lambda: (0, 0)),
            pl.BlockSpec((k, k), lambda: (0, 0)),
            pl.BlockSpec((1, k), lambda: (0, 0)),
            pl.BlockSpec((k, m), lambda: (0, 0)),
            pl.BlockSpec((1, m), lambda: (0, 0)),
        ],
        out_specs=pl.BlockSpec((1, m), lambda: (0, 0)),
        out_shape=jax.ShapeDtypeStruct((1, m), jnp.float32),
    )(xv, w1, b1.reshape(1, k), w2, b2.reshape(1, m))


def _make_segsum(n_pad, e_pad, cf, n_chunks, blk_k):
    cpc = n_chunks // _NC
    rows128 = e_pad // 128
    rt = rows128 // _NT
    nblk = rt // blk_k
    zr = n_pad // _NT
    nz = zr // 32

    mesh = plsc.VectorSubcoreMesh(core_axis_name="c", subcore_axis_name="s")

    gs = max(g for g in (40, 32, 24, 16, 8) if rt % g == 0)
    ngrp = rt // gs
    ngp = gs // 2

    def body(tbl_flat, src2d, dst2d, zbig, ones_h, sums_out, cnt_out,
             idx_sg, idx_dg, idx_z, rows_a, rows_b, bz_v,
             sem, sem_a, sem_b, acc):
        c = lax.axis_index("c")
        s = lax.axis_index("s")
        it16 = lax.iota(jnp.int32, 16)

        for l in range(8):
            idx_sg[gs, pl.ds(l * 16, 16)] = jnp.zeros((16,), jnp.int32)

        def zero_rows():
            pltpu.sync_copy(zbig, bz_v)
            for t in range(nz):
                base = s * zr + t * 32
                idx_z[0, pl.ds(0, 16)] = base + it16
                idx_z[0, pl.ds(16, 16)] = base + 16 + it16
                pltpu.sync_copy(bz_v, acc.at[idx_z.at[0]])

        def fire(j, rows_i, sem_i):
            pltpu.async_copy(tbl_flat.at[idx_sg.at[j]], rows_i.at[0], sem_i)

        def drain(rows_i, sem_i):
            pltpu.make_async_copy(
                tbl_flat.at[pl.ds(0, 128)], rows_i.at[0], sem_i).wait()

        for k in range(cpc):
            g = c * cpc + k
            zero_rows()
            plsc.subcore_barrier()

            for grp in range(ngrp):
                base_row = s * rt + grp * gs
                pltpu.sync_copy(
                    src2d.at[pl.ds(g * rows128 + base_row, gs)],
                    idx_sg.at[pl.ds(0, gs)])
                pltpu.sync_copy(dst2d.at[pl.ds(base_row, gs)],
                                idx_dg.at[pl.ds(0, gs)])
                fire(0, rows_a, sem_a)

                def blk(i2, carry):
                    j0 = 2 * i2
                    fire(j0 + 1, rows_b, sem_b)
                    drain(rows_a, sem_a)
                    pltpu.sync_copy(rows_a.at[0], acc.at[idx_dg.at[j0]],
                                    add=True)
                    fire(j0 + 2, rows_a, sem_a)
                    drain(rows_b, sem_b)
                    pltpu.sync_copy(rows_b.at[0], acc.at[idx_dg.at[j0 + 1]],
                                    add=True)
                    return carry

                lax.fori_loop(0, ngp, blk, 0)
                drain(rows_a, sem_a)
            plsc.subcore_barrier()

            for t in range(nz):
                base = s * zr + t * 32
                pltpu.async_copy(acc.at[pl.ds(base, 32)], bz_v, sem).wait()
                pltpu.sync_copy(bz_v, sums_out.at[pl.ds(g * n_pad + base, 32)])

        zero_rows()
        pltpu.sync_copy(ones_h, rows_a.at[0])
        plsc.subcore_barrier()
        for grp in range(ngrp):
            base_row = s * rt + grp * gs
            pltpu.sync_copy(dst2d.at[pl.ds(base_row, gs)],
                            idx_dg.at[pl.ds(0, gs)])

            def cblk(i2, carry):
                cp1 = pltpu.async_copy(rows_a.at[0], acc.at[idx_dg.at[2 * i2]],
                                       sem_a, add=True)
                cp2 = pltpu.async_copy(rows_a.at[0],
                                       acc.at[idx_dg.at[2 * i2 + 1]],
                                       sem_b, add=True)
                cp1.wait()
                cp2.wait()
                return carry

            lax.fori_loop(0, ngp, cblk, 0)
        plsc.subcore_barrier()
        for t in range(nz):
            base = s * zr + t * 32
            pltpu.async_copy(acc.at[pl.ds(base, 32)], bz_v, sem).wait()
            pltpu.sync_copy(bz_v, cnt_out.at[pl.ds(c * n_pad + base, 32)])

    return pl.kernel(
        body,
        out_type=(
            jax.ShapeDtypeStruct((n_chunks * n_pad, cf), jnp.float32),
            jax.ShapeDtypeStruct((2 * n_pad, cf), jnp.float32),
        ),
        mesh=mesh,
        scratch_types=[
            pltpu.VMEM((gs + 1, 128), jnp.int32),
            pltpu.VMEM((gs, 128), jnp.int32),
            pltpu.VMEM((1, 32), jnp.int32),
            pltpu.VMEM((1, 128, cf), jnp.float32),
            pltpu.VMEM((1, 128, cf), jnp.float32),
            pltpu.VMEM((32, cf), jnp.float32),
            pltpu.SemaphoreType.DMA,
            pltpu.SemaphoreType.DMA,
            pltpu.SemaphoreType.DMA,
            pltpu.VMEM_SHARED((n_pad, cf), jnp.float32),
        ],
    )


def _segment_mean_inputs(table, src, dst, n_nodes, n_pad, cf, n_chunks, blk_k):
    e = src.shape[0]
    unit = 128 * 128
    e_pad = ((e + unit - 1) // unit) * unit
    npd = e_pad - e
    pad_src = (jnp.arange(npd, dtype=jnp.int32) * 7) % n_nodes
    pad_dst = n_nodes + (jnp.arange(npd, dtype=jnp.int32) % (n_pad - n_nodes))
    src_p = jnp.concatenate([src, pad_src])
    dst_p = jnp.concatenate([dst, pad_dst])
    offs = (jnp.arange(n_chunks, dtype=jnp.int32) * n_pad)[:, None]
    src2d = (src_p[None, :] + offs).reshape(n_chunks * e_pad // 128, 128)
    dst2d = dst_p.reshape(e_pad // 128, 128)
    src2d = jnp.concatenate([src2d, jnp.zeros((1, 128), jnp.int32)])
    dst2d = jnp.concatenate([dst2d, jnp.full((1, 128), n_nodes, jnp.int32)])
    tbl_p = jnp.concatenate(
        [table, jnp.zeros((n_pad - n_nodes, table.shape[1]), jnp.float32)])
    tbl_flat = tbl_p.reshape(n_pad, n_chunks, cf).transpose(1, 0, 2)
    tbl_flat = tbl_flat.reshape(n_chunks * n_pad, cf)
    zbig = jnp.zeros((32, cf), jnp.float32)
    ones_h = jnp.ones((128, cf), jnp.float32)
    fn = _make_segsum(n_pad, e_pad, cf, n_chunks, blk_k)
    sums_flat, cnt_flat = fn(tbl_flat, src2d, dst2d, zbig, ones_h)
    return sums_flat.reshape(n_chunks, n_pad, cf), cnt_flat.reshape(2, n_pad, cf)


def kernel(x, params, edge_index_diff, feats_size_list):
    p = params
    ei0, ei1 = edge_index_diff

    y = _mm_leaky(x, p['l0_w'], p['l0_b'], 1000)

    n0 = _F0 + _F1
    n0_pad = 9728
    sums0, cnt0 = _segment_mean_inputs(
        y[:n0], ei0[0], ei0[1], n0, n0_pad, cf=128, n_chunks=4, blk_k=1)
    g0 = _conv_post(sums0, cnt0, p['conv0_w'], p['conv0_b'],
                    p['ln0_g'], p['ln0_b'], 608)

    n1 = _F1 + _F2
    n1_pad = 2560
    xx1 = jnp.concatenate([g0[_F0:n0], y[n0:]], axis=0)
    sums1, cnt1 = _segment_mean_inputs(
        xx1, ei1[0] - _F0, ei1[1] - _F0, n1, n1_pad, cf=128, n_chunks=4,
        blk_k=1)
    g1 = _conv_post(sums1, cnt1, p['conv1_w'], p['conv1_b'],
                    p['ln1_g'], p['ln1_b'], 2560)

    x0p = jnp.concatenate([y[:_F0], jnp.zeros((7680 - _F0, _H), jnp.float32)])
    specs = [(x0p, _F0, 512), (g0[_F0:n0], _F1, 400), (g1[_F1:n1], _F2, 500)]
    at_ = []
    xv_list = []
    for i, (xi, n_valid, blk) in enumerate(specs):
        xs, lg = _att_a(xi, p['attl1_%d_w' % i], p['attl1_%d_b' % i],
                        p['att1_%d_w' % i], p['att1_%d_b' % i],
                        p['att2_%d_w' % i], p['att2_%d_b' % i],
                        p['att3_%d_w' % i], p['att3_%d_b' % i], blk)
        a, xv = _att_b(lg, xs, blk, n_valid)
        at_.append(a[:n_valid].T)
        xv_list.append(xv)

    xv = jnp.concatenate(xv_list, axis=-1)
    out = _head(xv, p['llast_w'], p['llast_b'], p['lcla_w'], p['lcla_b'])
    return out, at_


# speedup vs baseline: 1.7796x; 1.7796x over previous
import functools

import jax
import jax.numpy as jnp
from jax import lax
from jax.experimental import pallas as pl
from jax.experimental.pallas import tpu as pltpu
from jax.experimental.pallas import tpu_sc as plsc

_F0, _F1, _F2 = 7500, 2000, 500
_H = 512

_NC = 2
_NT = 16


def _leaky(v):
    return jnp.where(v > 0, v, 0.01 * v)


def _mm_leaky_body(x_ref, w_ref, b_ref, o_ref):
    y = jnp.dot(x_ref[...], w_ref[...], preferred_element_type=jnp.float32)
    o_ref[...] = _leaky(y + b_ref[...])


def _mm_leaky(x, w, b, block_r):
    n, k = x.shape
    m = w.shape[1]
    return pl.pallas_call(
        _mm_leaky_body,
        grid=(n // block_r,),
        in_specs=[
            pl.BlockSpec((block_r, k), lambda i: (i, 0)),
            pl.BlockSpec((k, m), lambda i: (0, 0)),
            pl.BlockSpec((1, m), lambda i: (0, 0)),
        ],
        out_specs=pl.BlockSpec((block_r, m), lambda i: (i, 0)),
        out_shape=jax.ShapeDtypeStruct((n, m), jnp.float32),
    )(x, w, b.reshape(1, m))


def _conv_post_body(n_chunks, block_r, s_ref, c_ref, w_ref, b_ref, g_ref, bt_ref, o_ref):
    s = s_ref[...]
    agg = jnp.concatenate([s[j] for j in range(n_chunks)], axis=-1)
    cp = c_ref[...]
    cnt = jnp.clip((cp[0, :, :1] + cp[1, :, :1]), 1.0, None)
    h = jnp.dot(agg / cnt, w_ref[...], preferred_element_type=jnp.float32)
    h = h + b_ref[...]
    mu = jnp.mean(h, axis=-1, keepdims=True)
    var = jnp.mean((h - mu) ** 2, axis=-1, keepdims=True)
    h = (h - mu) / jnp.sqrt(var + 1e-5) * g_ref[...] + bt_ref[...]
    o_ref[...] = _leaky(h)


def _conv_post(sums, cnt, w, b, g, bt, block_r):
    n_rows = sums.shape[1]
    n_chunks, _, cf = sums.shape
    m = w.shape[1]
    return pl.pallas_call(
        functools.partial(_conv_post_body, n_chunks, block_r),
        grid=(n_rows // block_r,),
        in_specs=[
            pl.BlockSpec((n_chunks, block_r, cf), lambda i: (0, i, 0)),
            pl.BlockSpec((2, block_r, cf), lambda i: (0, i, 0)),
            pl.BlockSpec((_H, m), lambda i: (0, 0)),
            pl.BlockSpec((1, m), lambda i: (0, 0)),
            pl.BlockSpec((1, m), lambda i: (0, 0)),
            pl.BlockSpec((1, m), lambda i: (0, 0)),
        ],
        out_specs=pl.BlockSpec((block_r, m), lambda i: (i, 0)),
        out_shape=jax.ShapeDtypeStruct((n_rows, m), jnp.float32),
    )(sums, cnt, w, b.reshape(1, m), g.reshape(1, m), bt.reshape(1, m))


def _att_a_body(x_ref, wl_ref, bl_ref, w1_ref, b1_ref, w2_ref, b2_ref,
                w3_ref, b3_ref, xs_ref, lg_ref):
    xs = _leaky(jnp.dot(x_ref[...], wl_ref[...],
                        preferred_element_type=jnp.float32) + bl_ref[...])
    a1 = jnp.tanh(jnp.dot(xs, w1_ref[...],
                          preferred_element_type=jnp.float32) + b1_ref[...])
    a2 = jax.nn.sigmoid(jnp.dot(xs, w2_ref[...],
                                preferred_element_type=jnp.float32) + b2_ref[...])
    lg = jnp.dot(a1 * a2, w3_ref[...],
                 preferred_element_type=jnp.float32) + b3_ref[...]
    xs_ref[...] = xs
    lg_ref[...] = lg


def _att_a(xin, wl, bl, w1, b1, w2, b2, w3, b3, block_r):
    n = xin.shape[0]
    return pl.pallas_call(
        _att_a_body,
        grid=(n // block_r,),
        in_specs=[
            pl.BlockSpec((block_r, _H), lambda i: (i, 0)),
            pl.BlockSpec((_H, _H), lambda i: (0, 0)),
            pl.BlockSpec((1, _H), lambda i: (0, 0)),
            pl.BlockSpec((_H, _H), lambda i: (0, 0)),
            pl.BlockSpec((1, _H), lambda i: (0, 0)),
            pl.BlockSpec((_H, _H), lambda i: (0, 0)),
            pl.BlockSpec((1, _H), lambda i: (0, 0)),
            pl.BlockSpec((_H, 1), lambda i: (0, 0)),
            pl.BlockSpec((1, 1), lambda i: (0, 0)),
        ],
        out_specs=[
            pl.BlockSpec((block_r, _H), lambda i: (i, 0)),
            pl.BlockSpec((block_r, 1), lambda i: (i, 0)),
        ],
        out_shape=[
            jax.ShapeDtypeStruct((n, _H), jnp.float32),
            jax.ShapeDtypeStruct((n, 1), jnp.float32),
        ],
    )(xin, wl, bl.reshape(1, _H), w1, b1.reshape(1, _H), w2, b2.reshape(1, _H),
      w3, b3.reshape(1, 1))


def _att_b_body(block_r, n_valid, lg_ref, xs_ref, a_ref, xv_ref):
    i = pl.program_id(0)
    lg = lg_ref[...]
    if n_valid != lg.shape[0]:
        mask = lax.broadcasted_iota(jnp.int32, lg.shape, 0) < n_valid
        lg = jnp.where(mask, lg, -1e30)
    m = jnp.max(lg)
    den = jnp.sum(jnp.exp(lg - m))
    lgb = lg_ref[pl.ds(i * block_r, block_r), :]
    if n_valid != lg.shape[0]:
        bmask = (lax.broadcasted_iota(jnp.int32, (block_r, 1), 0)
                 + i * block_r) < n_valid
        lgb = jnp.where(bmask, lgb, -1e30)
    eb = jnp.exp(lgb - m) / den
    a_ref[...] = eb
    contrib = lax.dot_general(eb, xs_ref[...], (((0,), (0,)), ((), ())),
                              preferred_element_type=jnp.float32)

    @pl.when(i == 0)
    def _():
        xv_ref[...] = jnp.zeros_like(xv_ref)

    xv_ref[...] += contrib


def _att_b(lg, xs, block_r, n_valid):
    n = xs.shape[0]
    return pl.pallas_call(
        functools.partial(_att_b_body, block_r, n_valid),
        grid=(n // block_r,),
        in_specs=[
            pl.BlockSpec((n, 1), lambda i: (0, 0)),
            pl.BlockSpec((block_r, _H), lambda i: (i, 0)),
        ],
        out_specs=[
            pl.BlockSpec((block_r, 1), lambda i: (i, 0)),
            pl.BlockSpec((1, _H), lambda i: (0, 0)),
        ],
        out_shape=[
            jax.ShapeDtypeStruct((n, 1), jnp.float32),
            jax.ShapeDtypeStruct((1, _H), jnp.float32),
        ],
    )(lg, xs)


def _head_body(xv_ref, w1_ref, b1_ref, w2_ref, b2_ref, o_ref):
    h = _leaky(jnp.dot(xv_ref[...], w1_ref[...],
                       preferred_element_type=jnp.float32) + b1_ref[...])
    o_ref[...] = jax.nn.sigmoid(
        jnp.dot(h, w2_ref[...], preferred_element_type=jnp.float32) + b2_ref[...])


def _head(xv, w1, b1, w2, b2):
    k = w1.shape[0]
    m = w2.shape[1]
    return pl.pallas_call(
        _head_body,
        in_specs=[
            pl.BlockSpec((1, k), lambda: (0, 0)),
            pl.BlockSpec((k, k), lambda: (0, 0)),
            pl.BlockSpec((1, k), lambda: (0, 0)),
            pl.BlockSpec((k, m), lambda: (0, 0)),
            pl.BlockSpec((1, m), lambda: (0, 0)),
        ],
        out_specs=pl.BlockSpec((1, m), lambda: (0, 0)),
        out_shape=jax.ShapeDtypeStruct((1, m), jnp.float32),
    )(xv, w1, b1.reshape(1, k), w2, b2.reshape(1, m))


def _make_segsum(n_pad, e_pad, cf, n_chunks, blk_k):
    cpc = n_chunks // _NC
    rows128 = e_pad // 128
    rt = rows128 // _NT
    nblk = rt // blk_k
    zr = n_pad // _NT
    nz = zr // 32

    mesh = plsc.VectorSubcoreMesh(core_axis_name="c", subcore_axis_name="s")

    def body(tbl_flat, src2d, dst2d, zbig, ones_h, sums_out, cnt_out,
             idx_sa, idx_da, idx_sb, idx_db, idx_z, rows_a, rows_b,
             sem, sem_a, sem_b, acc):
        c = lax.axis_index("c")
        s = lax.axis_index("s")
        it16 = lax.iota(jnp.int32, 16)

        bases = list(range(0, zr - 127, 128))
        if zr % 128:
            bases.append(zr - 128)

        def zero_rows():
            pltpu.sync_copy(zbig, rows_a.at[0])
            cps = []
            for t, b0 in enumerate(bases):
                base = s * zr + b0
                for l in range(8):
                    idx_z[t, pl.ds(l * 16, 16)] = base + l * 16 + it16
                cps.append(pltpu.async_copy(rows_a.at[0], acc.at[idx_z.at[t]],
                                            sem))
            for cp in cps:
                cp.wait()

        def copy_out(dst_hbm, row0):
            rds = [pltpu.async_copy(acc.at[pl.ds(s * zr + b0, 128)],
                                    (rows_a if t % 2 == 0 else rows_b).at[0],
                                    sem_a if t % 2 == 0 else sem_b)
                   for t, b0 in enumerate(bases[:1])]
            for t, b0 in enumerate(bases):
                if t + 1 < len(bases):
                    rds.append(pltpu.async_copy(
                        acc.at[pl.ds(s * zr + bases[t + 1], 128)],
                        (rows_a if (t + 1) % 2 == 0 else rows_b).at[0],
                        sem_a if (t + 1) % 2 == 0 else sem_b))
                rds[t].wait()
                pltpu.sync_copy((rows_a if t % 2 == 0 else rows_b).at[0],
                                dst_hbm.at[pl.ds(row0 + s * zr + b0, 128)])

        def load_fire(g, r, idx_si, idx_di, rows_i, sem_i):
            pltpu.sync_copy(src2d.at[pl.ds(g * rows128 + r, 1)], idx_si)
            pltpu.sync_copy(dst2d.at[pl.ds(r, 1)], idx_di)
            pltpu.async_copy(tbl_flat.at[idx_si.at[0]], rows_i.at[0], sem_i)

        def drain(rows_i, sem_i):
            pltpu.make_async_copy(
                tbl_flat.at[pl.ds(0, 128)], rows_i.at[0], sem_i).wait()

        for k in range(cpc):
            g = c * cpc + k
            zero_rows()
            plsc.subcore_barrier()

            load_fire(g, s * rt, idx_sa, idx_da, rows_a, sem_a)

            def blk(i2, carry):
                r0 = s * rt + 2 * i2
                load_fire(g, r0 + 1, idx_sb, idx_db, rows_b, sem_b)
                drain(rows_a, sem_a)
                pltpu.sync_copy(rows_a.at[0], acc.at[idx_da.at[0]], add=True)
                load_fire(g, r0 + 2, idx_sa, idx_da, rows_a, sem_a)
                drain(rows_b, sem_b)
                pltpu.sync_copy(rows_b.at[0], acc.at[idx_db.at[0]], add=True)
                return carry

            lax.fori_loop(0, nblk // 2, blk, 0)
            drain(rows_a, sem_a)
            plsc.subcore_barrier()

            copy_out(sums_out, g * n_pad)

        zero_rows()
        pltpu.sync_copy(ones_h, rows_a.at[0])
        plsc.subcore_barrier()
        rt2 = rows128 // (2 * _NT)

        def cblk(i2, carry):
            r0 = c * (rows128 // 2) + s * rt2 + 2 * i2
            pltpu.sync_copy(dst2d.at[pl.ds(r0, 1)], idx_da)
            pltpu.sync_copy(dst2d.at[pl.ds(r0 + 1, 1)], idx_db)
            cp1 = pltpu.async_copy(rows_a.at[0], acc.at[idx_da.at[0]], sem_a,
                                   add=True)
            cp2 = pltpu.async_copy(rows_a.at[0], acc.at[idx_db.at[0]], sem_b,
                                   add=True)
            cp1.wait()
            cp2.wait()
            return carry

        lax.fori_loop(0, rt2 // 2, cblk, 0)
        plsc.subcore_barrier()
        copy_out(cnt_out, c * n_pad)

    return pl.kernel(
        body,
        out_type=(
            jax.ShapeDtypeStruct((n_chunks * n_pad, cf), jnp.float32),
            jax.ShapeDtypeStruct((2 * n_pad, cf), jnp.float32),
        ),
        mesh=mesh,
        scratch_types=[
            pltpu.VMEM((1, 128), jnp.int32),
            pltpu.VMEM((1, 128), jnp.int32),
            pltpu.VMEM((1, 128), jnp.int32),
            pltpu.VMEM((1, 128), jnp.int32),
            pltpu.VMEM((8, 128), jnp.int32),
            pltpu.VMEM((1, 128, cf), jnp.float32),
            pltpu.VMEM((1, 128, cf), jnp.float32),
            pltpu.SemaphoreType.DMA,
            pltpu.SemaphoreType.DMA,
            pltpu.SemaphoreType.DMA,
            pltpu.VMEM_SHARED((n_pad, cf), jnp.float32),
        ],
    )


def _segment_mean_inputs(table, src, dst, n_nodes, n_pad, cf, n_chunks, blk_k):
    e = src.shape[0]
    unit = 128 * _NT * blk_k
    e_pad = ((e + unit - 1) // unit) * unit
    src_p = jnp.concatenate([src, jnp.zeros((e_pad - e,), jnp.int32)])
    dst_p = jnp.concatenate([dst, jnp.full((e_pad - e,), n_nodes, jnp.int32)])
    offs = (jnp.arange(n_chunks, dtype=jnp.int32) * n_pad)[:, None]
    src2d = (src_p[None, :] + offs).reshape(n_chunks * e_pad // 128, 128)
    dst2d = dst_p.reshape(e_pad // 128, 128)
    src2d = jnp.concatenate([src2d, jnp.zeros((1, 128), jnp.int32)])
    dst2d = jnp.concatenate([dst2d, jnp.full((1, 128), n_nodes, jnp.int32)])
    tbl_p = jnp.concatenate(
        [table, jnp.zeros((n_pad - n_nodes, table.shape[1]), jnp.float32)])
    tbl_flat = tbl_p.reshape(n_pad, n_chunks, cf).transpose(1, 0, 2)
    tbl_flat = tbl_flat.reshape(n_chunks * n_pad, cf)
    zbig = jnp.zeros((128, cf), jnp.float32)
    ones_h = jnp.ones((128, cf), jnp.float32)
    fn = _make_segsum(n_pad, e_pad, cf, n_chunks, blk_k)
    sums_flat, cnt_flat = fn(tbl_flat, src2d, dst2d, zbig, ones_h)
    return sums_flat.reshape(n_chunks, n_pad, cf), cnt_flat.reshape(2, n_pad, cf)


def kernel(x, params, edge_index_diff, feats_size_list):
    p = params
    ei0, ei1 = edge_index_diff

    y = _mm_leaky(x, p['l0_w'], p['l0_b'], 1000)

    n0 = _F0 + _F1
    n0_pad = 9728
    sums0, cnt0 = _segment_mean_inputs(
        y[:n0], ei0[0], ei0[1], n0, n0_pad, cf=128, n_chunks=4, blk_k=1)
    g0 = _conv_post(sums0, cnt0, p['conv0_w'], p['conv0_b'],
                    p['ln0_g'], p['ln0_b'], 608)

    n1 = _F1 + _F2
    n1_pad = 2560
    xx1 = jnp.concatenate([g0[_F0:n0], y[n0:]], axis=0)
    sums1, cnt1 = _segment_mean_inputs(
        xx1, ei1[0] - _F0, ei1[1] - _F0, n1, n1_pad, cf=128, n_chunks=4,
        blk_k=1)
    g1 = _conv_post(sums1, cnt1, p['conv1_w'], p['conv1_b'],
                    p['ln1_g'], p['ln1_b'], 2560)

    x0p = jnp.concatenate([y[:_F0], jnp.zeros((7680 - _F0, _H), jnp.float32)])
    specs = [(x0p, _F0, 512), (g0[_F0:n0], _F1, 400), (g1[_F1:n1], _F2, 500)]
    at_ = []
    xv_list = []
    for i, (xi, n_valid, blk) in enumerate(specs):
        xs, lg = _att_a(xi, p['attl1_%d_w' % i], p['attl1_%d_b' % i],
                        p['att1_%d_w' % i], p['att1_%d_b' % i],
                        p['att2_%d_w' % i], p['att2_%d_b' % i],
                        p['att3_%d_w' % i], p['att3_%d_b' % i], blk)
        a, xv = _att_b(lg, xs, blk, n_valid)
        at_.append(a[:n_valid].T)
        xv_list.append(xv)

    xv = jnp.concatenate(xv_list, axis=-1)
    out = _head(xv, p['llast_w'], p['llast_b'], p['lcla_w'], p['lcla_b'])
    return out, at_


# speedup vs baseline: 2.6491x; 1.4886x over previous
import functools

import jax
import jax.numpy as jnp
from jax import lax
from jax.experimental import pallas as pl
from jax.experimental.pallas import tpu as pltpu
from jax.experimental.pallas import tpu_sc as plsc

_F0, _F1, _F2 = 7500, 2000, 500
_H = 512

_NC = 2
_NT = 16


def _leaky(v):
    return jnp.where(v > 0, v, 0.01 * v)


def _mm_leaky_body(x_ref, w_ref, b_ref, o_ref):
    y = jnp.dot(x_ref[...], w_ref[...], preferred_element_type=jnp.float32)
    o_ref[...] = _leaky(y + b_ref[...])


def _mm_leaky(x, w, b, block_r):
    n, k = x.shape
    m = w.shape[1]
    return pl.pallas_call(
        _mm_leaky_body,
        grid=(n // block_r,),
        in_specs=[
            pl.BlockSpec((block_r, k), lambda i: (i, 0)),
            pl.BlockSpec((k, m), lambda i: (0, 0)),
            pl.BlockSpec((1, m), lambda i: (0, 0)),
        ],
        out_specs=pl.BlockSpec((block_r, m), lambda i: (i, 0)),
        out_shape=jax.ShapeDtypeStruct((n, m), jnp.float32),
    )(x, w, b.reshape(1, m))


def _conv_post_body(n_chunks, block_r, s_ref, c_ref, w_ref, b_ref, g_ref, bt_ref, o_ref):
    s = s_ref[...]
    agg = jnp.concatenate([s[j] for j in range(n_chunks)], axis=-1)
    cp = c_ref[...]
    cnt = jnp.clip((cp[0, :, :1] + cp[1, :, :1]) * 0.5, 1.0, None)
    h = jnp.dot(agg / cnt, w_ref[...], preferred_element_type=jnp.float32)
    h = h + b_ref[...]
    mu = jnp.mean(h, axis=-1, keepdims=True)
    var = jnp.mean((h - mu) ** 2, axis=-1, keepdims=True)
    h = (h - mu) / jnp.sqrt(var + 1e-5) * g_ref[...] + bt_ref[...]
    o_ref[...] = _leaky(h)


def _conv_post(sums, cnt, w, b, g, bt, block_r):
    n_rows = sums.shape[1]
    n_chunks, _, cf = sums.shape
    m = w.shape[1]
    return pl.pallas_call(
        functools.partial(_conv_post_body, n_chunks, block_r),
        grid=(n_rows // block_r,),
        in_specs=[
            pl.BlockSpec((n_chunks, block_r, cf), lambda i: (0, i, 0)),
            pl.BlockSpec((2, block_r, cf), lambda i: (0, i, 0)),
            pl.BlockSpec((_H, m), lambda i: (0, 0)),
            pl.BlockSpec((1, m), lambda i: (0, 0)),
            pl.BlockSpec((1, m), lambda i: (0, 0)),
            pl.BlockSpec((1, m), lambda i: (0, 0)),
        ],
        out_specs=pl.BlockSpec((block_r, m), lambda i: (i, 0)),
        out_shape=jax.ShapeDtypeStruct((n_rows, m), jnp.float32),
    )(sums, cnt, w, b.reshape(1, m), g.reshape(1, m), bt.reshape(1, m))


def _att_a_body(x_ref, wl_ref, bl_ref, w1_ref, b1_ref, w2_ref, b2_ref,
                w3_ref, b3_ref, xs_ref, lg_ref):
    xs = _leaky(jnp.dot(x_ref[...], wl_ref[...],
                        preferred_element_type=jnp.float32) + bl_ref[...])
    a1 = jnp.tanh(jnp.dot(xs, w1_ref[...],
                          preferred_element_type=jnp.float32) + b1_ref[...])
    a2 = jax.nn.sigmoid(jnp.dot(xs, w2_ref[...],
                                preferred_element_type=jnp.float32) + b2_ref[...])
    lg = jnp.dot(a1 * a2, w3_ref[...],
                 preferred_element_type=jnp.float32) + b3_ref[...]
    xs_ref[...] = xs
    lg_ref[...] = lg


def _att_a(xin, wl, bl, w1, b1, w2, b2, w3, b3, block_r):
    n = xin.shape[0]
    return pl.pallas_call(
        _att_a_body,
        grid=(n // block_r,),
        in_specs=[
            pl.BlockSpec((block_r, _H), lambda i: (i, 0)),
            pl.BlockSpec((_H, _H), lambda i: (0, 0)),
            pl.BlockSpec((1, _H), lambda i: (0, 0)),
            pl.BlockSpec((_H, _H), lambda i: (0, 0)),
            pl.BlockSpec((1, _H), lambda i: (0, 0)),
            pl.BlockSpec((_H, _H), lambda i: (0, 0)),
            pl.BlockSpec((1, _H), lambda i: (0, 0)),
            pl.BlockSpec((_H, 1), lambda i: (0, 0)),
            pl.BlockSpec((1, 1), lambda i: (0, 0)),
        ],
        out_specs=[
            pl.BlockSpec((block_r, _H), lambda i: (i, 0)),
            pl.BlockSpec((block_r, 1), lambda i: (i, 0)),
        ],
        out_shape=[
            jax.ShapeDtypeStruct((n, _H), jnp.float32),
            jax.ShapeDtypeStruct((n, 1), jnp.float32),
        ],
    )(xin, wl, bl.reshape(1, _H), w1, b1.reshape(1, _H), w2, b2.reshape(1, _H),
      w3, b3.reshape(1, 1))


def _att_b_body(block_r, n_valid, lg_ref, xs_ref, a_ref, xv_ref):
    i = pl.program_id(0)
    lg = lg_ref[...]
    if n_valid != lg.shape[0]:
        mask = lax.broadcasted_iota(jnp.int32, lg.shape, 0) < n_valid
        lg = jnp.where(mask, lg, -1e30)
    m = jnp.max(lg)
    den = jnp.sum(jnp.exp(lg - m))
    lgb = lg_ref[pl.ds(i * block_r, block_r), :]
    if n_valid != lg.shape[0]:
        bmask = (lax.broadcasted_iota(jnp.int32, (block_r, 1), 0)
                 + i * block_r) < n_valid
        lgb = jnp.where(bmask, lgb, -1e30)
    eb = jnp.exp(lgb - m) / den
    a_ref[...] = eb
    contrib = lax.dot_general(eb, xs_ref[...], (((0,), (0,)), ((), ())),
                              preferred_element_type=jnp.float32)

    @pl.when(i == 0)
    def _():
        xv_ref[...] = jnp.zeros_like(xv_ref)

    xv_ref[...] += contrib


def _att_b(lg, xs, block_r, n_valid):
    n = xs.shape[0]
    return pl.pallas_call(
        functools.partial(_att_b_body, block_r, n_valid),
        grid=(n // block_r,),
        in_specs=[
            pl.BlockSpec((n, 1), lambda i: (0, 0)),
            pl.BlockSpec((block_r, _H), lambda i: (i, 0)),
        ],
        out_specs=[
            pl.BlockSpec((block_r, 1), lambda i: (i, 0)),
            pl.BlockSpec((1, _H), lambda i: (0, 0)),
        ],
        out_shape=[
            jax.ShapeDtypeStruct((n, 1), jnp.float32),
            jax.ShapeDtypeStruct((1, _H), jnp.float32),
        ],
    )(lg, xs)


def _head_body(xv_ref, w1_ref, b1_ref, w2_ref, b2_ref, o_ref):
    h = _leaky(jnp.dot(xv_ref[...], w1_ref[...],
                       preferred_element_type=jnp.float32) + b1_ref[...])
    o_ref[...] = jax.nn.sigmoid(
        jnp.dot(h, w2_ref[...], preferred_element_type=jnp.float32) + b2_ref[...])


def _head(xv, w1, b1, w2, b2):
    k = w1.shape[0]
    m = w2.shape[1]
    return pl.pallas_call(
        _head_body,
        in_specs=[
            pl.BlockSpec((1, k), lambda: (0, 0)),
            pl.BlockSpec((k, k), lambda: (0, 0)),
            pl.BlockSpec((1, k), lambda: (0, 0)),
            pl.BlockSpec((k, m), lambda: (0, 0)),
            pl.BlockSpec((1, m), lambda: (0, 0)),
        ],
        out_specs=pl.BlockSpec((1, m), lambda: (0, 0)),
        out_shape=jax.ShapeDtypeStruct((1, m), jnp.float32),
    )(xv, w1, b1.reshape(1, k), w2, b2.reshape(1, m))


def _make_segsum(n_pad, e_pad, cf, n_chunks, blk_k):
    cpc = n_chunks // _NC
    rows128 = e_pad // 128
    rt = rows128 // _NT
    gs = max(g for g in (40, 32, 24, 16, 8) if rt % g == 0)
    ngrp = rt // gs
    zr = n_pad // _NT

    mesh = plsc.VectorSubcoreMesh(core_axis_name="c", subcore_axis_name="s")

    def body(tbl_flat, src2d, dst2d, zbig, ones_h, sums_out, cnt_out,
             idx_sg, idx_dg, idx_z, rows_a, rows_b,
             sem, sem_a, sem_b, acc):
        c = lax.axis_index("c")
        s = lax.axis_index("s")
        it16 = lax.iota(jnp.int32, 16)

        bases = list(range(0, zr - 127, 128))
        if zr % 128:
            bases.append(zr - 128)

        def zero_rows():
            pltpu.sync_copy(zbig, rows_a.at[0])
            cps = []
            for t, b0 in enumerate(bases):
                base = s * zr + b0
                for l in range(8):
                    idx_z[t, pl.ds(l * 16, 16)] = base + l * 16 + it16
                cps.append(pltpu.async_copy(rows_a.at[0], acc.at[idx_z.at[t]],
                                            sem))
            for cp in cps:
                cp.wait()

        def copy_out(dst_hbm, row0):
            rds = [pltpu.async_copy(acc.at[pl.ds(s * zr + b0, 128)],
                                    (rows_a if t % 2 == 0 else rows_b).at[0],
                                    sem_a if t % 2 == 0 else sem_b)
                   for t, b0 in enumerate(bases[:1])]
            for t, b0 in enumerate(bases):
                if t + 1 < len(bases):
                    rds.append(pltpu.async_copy(
                        acc.at[pl.ds(s * zr + bases[t + 1], 128)],
                        (rows_a if (t + 1) % 2 == 0 else rows_b).at[0],
                        sem_a if (t + 1) % 2 == 0 else sem_b))
                rds[t].wait()
                pltpu.sync_copy((rows_a if t % 2 == 0 else rows_b).at[0],
                                dst_hbm.at[pl.ds(row0 + s * zr + b0, 128)])

        def drain(rows_i, sem_i):
            pltpu.make_async_copy(
                tbl_flat.at[pl.ds(0, 128)], rows_i.at[0], sem_i).wait()

        def fire(j, rows_i, sem_i):
            pltpu.async_copy(tbl_flat.at[idx_sg.at[j]], rows_i.at[0], sem_i)

        for k in range(cpc):
            g = c * cpc + k
            zero_rows()
            plsc.subcore_barrier()

            for grp in range(ngrp):
                base_row = s * rt + grp * gs
                pltpu.sync_copy(src2d.at[pl.ds(g * rows128 + base_row, gs)],
                                idx_sg.at[pl.ds(0, gs)])
                pltpu.sync_copy(dst2d.at[pl.ds(base_row, gs)],
                                idx_dg.at[pl.ds(0, gs)])
                fire(0, rows_a, sem_a)
                for j in range(0, gs, 2):
                    fire(j + 1, rows_b, sem_b)
                    drain(rows_a, sem_a)
                    pltpu.sync_copy(rows_a.at[0], acc.at[idx_dg.at[j]],
                                    add=True)
                    if j + 2 < gs:
                        fire(j + 2, rows_a, sem_a)
                    drain(rows_b, sem_b)
                    pltpu.sync_copy(rows_b.at[0], acc.at[idx_dg.at[j + 1]],
                                    add=True)
            plsc.subcore_barrier()

            copy_out(sums_out, g * n_pad)

        zero_rows()
        pltpu.sync_copy(ones_h, rows_a.at[0])
        plsc.subcore_barrier()
        for grp in range(ngrp):
            base_row = s * rt + grp * gs
            pltpu.sync_copy(dst2d.at[pl.ds(base_row, gs)],
                            idx_dg.at[pl.ds(0, gs)])
            for j in range(0, gs, 2):
                cp1 = pltpu.async_copy(rows_a.at[0], acc.at[idx_dg.at[j]],
                                       sem_a, add=True)
                cp2 = pltpu.async_copy(rows_a.at[0], acc.at[idx_dg.at[j + 1]],
                                       sem_b, add=True)
                cp1.wait()
                cp2.wait()
        plsc.subcore_barrier()
        copy_out(cnt_out, c * n_pad)

    return pl.kernel(
        body,
        out_type=(
            jax.ShapeDtypeStruct((n_chunks * n_pad, cf), jnp.float32),
            jax.ShapeDtypeStruct((2 * n_pad, cf), jnp.float32),
        ),
        mesh=mesh,
        scratch_types=[
            pltpu.VMEM((gs, 128), jnp.int32),
            pltpu.VMEM((gs, 128), jnp.int32),
            pltpu.VMEM((8, 128), jnp.int32),
            pltpu.VMEM((1, 128, cf), jnp.float32),
            pltpu.VMEM((1, 128, cf), jnp.float32),
            pltpu.SemaphoreType.DMA,
            pltpu.SemaphoreType.DMA,
            pltpu.SemaphoreType.DMA,
            pltpu.VMEM_SHARED((n_pad, cf), jnp.float32),
        ],
    )


def _segment_mean_inputs(table, src, dst, n_nodes, n_pad, cf, n_chunks, blk_k):
    e = src.shape[0]
    unit = 128 * 128
    e_pad = ((e + unit - 1) // unit) * unit
    npd = e_pad - e
    pad_src = (jnp.arange(npd, dtype=jnp.int32) * 7) % n_nodes
    pad_dst = n_nodes + (jnp.arange(npd, dtype=jnp.int32) % (n_pad - n_nodes))
    src_p = jnp.concatenate([src, pad_src])
    dst_p = jnp.concatenate([dst, pad_dst])
    offs = (jnp.arange(n_chunks, dtype=jnp.int32) * n_pad)[:, None]
    src2d = (src_p[None, :] + offs).reshape(n_chunks * e_pad // 128, 128)
    dst2d = dst_p.reshape(e_pad // 128, 128)
    src2d = jnp.concatenate([src2d, jnp.zeros((1, 128), jnp.int32)])
    dst2d = jnp.concatenate([dst2d, jnp.full((1, 128), n_nodes, jnp.int32)])
    tbl_p = jnp.concatenate(
        [table, jnp.zeros((n_pad - n_nodes, table.shape[1]), jnp.float32)])
    tbl_flat = tbl_p.reshape(n_pad, n_chunks, cf).transpose(1, 0, 2)
    tbl_flat = tbl_flat.reshape(n_chunks * n_pad, cf)
    zbig = jnp.zeros((128, cf), jnp.float32)
    ones_h = jnp.ones((128, cf), jnp.float32)
    fn = _make_segsum(n_pad, e_pad, cf, n_chunks, blk_k)
    sums_flat, cnt_flat = fn(tbl_flat, src2d, dst2d, zbig, ones_h)
    return sums_flat.reshape(n_chunks, n_pad, cf), cnt_flat.reshape(2, n_pad, cf)


def kernel(x, params, edge_index_diff, feats_size_list):
    p = params
    ei0, ei1 = edge_index_diff

    y = _mm_leaky(x, p['l0_w'], p['l0_b'], 1000)

    n0 = _F0 + _F1
    n0_pad = 9728
    sums0, cnt0 = _segment_mean_inputs(
        y[:n0], ei0[0], ei0[1], n0, n0_pad, cf=128, n_chunks=4, blk_k=1)
    g0 = _conv_post(sums0, cnt0, p['conv0_w'], p['conv0_b'],
                    p['ln0_g'], p['ln0_b'], 608)

    n1 = _F1 + _F2
    n1_pad = 2560
    xx1 = jnp.concatenate([g0[_F0:n0], y[n0:]], axis=0)
    sums1, cnt1 = _segment_mean_inputs(
        xx1, ei1[0] - _F0, ei1[1] - _F0, n1, n1_pad, cf=128, n_chunks=4,
        blk_k=1)
    g1 = _conv_post(sums1, cnt1, p['conv1_w'], p['conv1_b'],
                    p['ln1_g'], p['ln1_b'], 2560)

    x0p = jnp.concatenate([y[:_F0], jnp.zeros((7680 - _F0, _H), jnp.float32)])
    specs = [(x0p, _F0, 512), (g0[_F0:n0], _F1, 400), (g1[_F1:n1], _F2, 500)]
    at_ = []
    xv_list = []
    for i, (xi, n_valid, blk) in enumerate(specs):
        xs, lg = _att_a(xi, p['attl1_%d_w' % i], p['attl1_%d_b' % i],
                        p['att1_%d_w' % i], p['att1_%d_b' % i],
                        p['att2_%d_w' % i], p['att2_%d_b' % i],
                        p['att3_%d_w' % i], p['att3_%d_b' % i], blk)
        a, xv = _att_b(lg, xs, blk, n_valid)
        at_.append(a[:n_valid].T)
        xv_list.append(xv)

    xv = jnp.concatenate(xv_list, axis=-1)
    out = _head(xv, p['llast_w'], p['llast_b'], p['lcla_w'], p['lcla_b'])
    return out, at_


# speedup vs baseline: 2.7863x; 1.0518x over previous
import functools

import jax
import jax.numpy as jnp
from jax import lax
from jax.experimental import pallas as pl
from jax.experimental.pallas import tpu as pltpu
from jax.experimental.pallas import tpu_sc as plsc

_F0, _F1, _F2 = 7500, 2000, 500
_H = 512

_NC = 2
_NT = 16


def _leaky(v):
    return jnp.where(v > 0, v, 0.01 * v)


def _mm_leaky_body(x_ref, w_ref, b_ref, o_ref):
    y = jnp.dot(x_ref[...], w_ref[...], preferred_element_type=jnp.float32)
    o_ref[...] = _leaky(y + b_ref[...])


def _mm_leaky(x, w, b, block_r):
    n, k = x.shape
    m = w.shape[1]
    return pl.pallas_call(
        _mm_leaky_body,
        grid=(n // block_r,),
        in_specs=[
            pl.BlockSpec((block_r, k), lambda i: (i, 0)),
            pl.BlockSpec((k, m), lambda i: (0, 0)),
            pl.BlockSpec((1, m), lambda i: (0, 0)),
        ],
        out_specs=pl.BlockSpec((block_r, m), lambda i: (i, 0)),
        out_shape=jax.ShapeDtypeStruct((n, m), jnp.float32),
    )(x, w, b.reshape(1, m))


def _conv_post_body(n_chunks, block_r, s_ref, c_ref, w_ref, b_ref, g_ref, bt_ref, o_ref):
    s = s_ref[...]
    agg = jnp.concatenate([s[j] for j in range(n_chunks)], axis=-1)
    cp = c_ref[...]
    cnt = jnp.clip(cp[0, :, :1] + cp[1, :, :1], 1.0, None)
    h = jnp.dot(agg / cnt, w_ref[...], preferred_element_type=jnp.float32)
    h = h + b_ref[...]
    mu = jnp.mean(h, axis=-1, keepdims=True)
    var = jnp.mean((h - mu) ** 2, axis=-1, keepdims=True)
    h = (h - mu) / jnp.sqrt(var + 1e-5) * g_ref[...] + bt_ref[...]
    o_ref[...] = _leaky(h)


def _conv_post(sums, cnt, w, b, g, bt, block_r):
    n_rows = sums.shape[1]
    n_chunks, _, cf = sums.shape
    m = w.shape[1]
    return pl.pallas_call(
        functools.partial(_conv_post_body, n_chunks, block_r),
        grid=(n_rows // block_r,),
        in_specs=[
            pl.BlockSpec((n_chunks, block_r, cf), lambda i: (0, i, 0)),
            pl.BlockSpec((2, block_r, cf), lambda i: (0, i, 0)),
            pl.BlockSpec((_H, m), lambda i: (0, 0)),
            pl.BlockSpec((1, m), lambda i: (0, 0)),
            pl.BlockSpec((1, m), lambda i: (0, 0)),
            pl.BlockSpec((1, m), lambda i: (0, 0)),
        ],
        out_specs=pl.BlockSpec((block_r, m), lambda i: (i, 0)),
        out_shape=jax.ShapeDtypeStruct((n_rows, m), jnp.float32),
    )(sums, cnt, w, b.reshape(1, m), g.reshape(1, m), bt.reshape(1, m))


def _att_a_body(x_ref, wl_ref, bl_ref, w1_ref, b1_ref, w2_ref, b2_ref,
                w3_ref, b3_ref, xs_ref, lg_ref):
    xs = _leaky(jnp.dot(x_ref[...], wl_ref[...],
                        preferred_element_type=jnp.float32) + bl_ref[...])
    a1 = jnp.tanh(jnp.dot(xs, w1_ref[...],
                          preferred_element_type=jnp.float32) + b1_ref[...])
    a2 = jax.nn.sigmoid(jnp.dot(xs, w2_ref[...],
                                preferred_element_type=jnp.float32) + b2_ref[...])
    lg = jnp.dot(a1 * a2, w3_ref[...],
                 preferred_element_type=jnp.float32) + b3_ref[...]
    xs_ref[...] = xs
    lg_ref[...] = lg


def _att_a(xin, wl, bl, w1, b1, w2, b2, w3, b3, block_r):
    n = xin.shape[0]
    return pl.pallas_call(
        _att_a_body,
        grid=(n // block_r,),
        in_specs=[
            pl.BlockSpec((block_r, _H), lambda i: (i, 0)),
            pl.BlockSpec((_H, _H), lambda i: (0, 0)),
            pl.BlockSpec((1, _H), lambda i: (0, 0)),
            pl.BlockSpec((_H, _H), lambda i: (0, 0)),
            pl.BlockSpec((1, _H), lambda i: (0, 0)),
            pl.BlockSpec((_H, _H), lambda i: (0, 0)),
            pl.BlockSpec((1, _H), lambda i: (0, 0)),
            pl.BlockSpec((_H, 1), lambda i: (0, 0)),
            pl.BlockSpec((1, 1), lambda i: (0, 0)),
        ],
        out_specs=[
            pl.BlockSpec((block_r, _H), lambda i: (i, 0)),
            pl.BlockSpec((block_r, 1), lambda i: (i, 0)),
        ],
        out_shape=[
            jax.ShapeDtypeStruct((n, _H), jnp.float32),
            jax.ShapeDtypeStruct((n, 1), jnp.float32),
        ],
    )(xin, wl, bl.reshape(1, _H), w1, b1.reshape(1, _H), w2, b2.reshape(1, _H),
      w3, b3.reshape(1, 1))


def _att_b_body(block_r, n_valid, lg_ref, xs_ref, a_ref, xv_ref):
    i = pl.program_id(0)
    lg = lg_ref[...]
    if n_valid != lg.shape[0]:
        mask = lax.broadcasted_iota(jnp.int32, lg.shape, 0) < n_valid
        lg = jnp.where(mask, lg, -1e30)
    m = jnp.max(lg)
    den = jnp.sum(jnp.exp(lg - m))
    lgb = lg_ref[pl.ds(i * block_r, block_r), :]
    if n_valid != lg.shape[0]:
        bmask = (lax.broadcasted_iota(jnp.int32, (block_r, 1), 0)
                 + i * block_r) < n_valid
        lgb = jnp.where(bmask, lgb, -1e30)
    eb = jnp.exp(lgb - m) / den
    a_ref[...] = eb
    contrib = lax.dot_general(eb, xs_ref[...], (((0,), (0,)), ((), ())),
                              preferred_element_type=jnp.float32)

    @pl.when(i == 0)
    def _():
        xv_ref[...] = jnp.zeros_like(xv_ref)

    xv_ref[...] += contrib


def _att_b(lg, xs, block_r, n_valid):
    n = xs.shape[0]
    return pl.pallas_call(
        functools.partial(_att_b_body, block_r, n_valid),
        grid=(n // block_r,),
        in_specs=[
            pl.BlockSpec((n, 1), lambda i: (0, 0)),
            pl.BlockSpec((block_r, _H), lambda i: (i, 0)),
        ],
        out_specs=[
            pl.BlockSpec((block_r, 1), lambda i: (i, 0)),
            pl.BlockSpec((1, _H), lambda i: (0, 0)),
        ],
        out_shape=[
            jax.ShapeDtypeStruct((n, 1), jnp.float32),
            jax.ShapeDtypeStruct((1, _H), jnp.float32),
        ],
    )(lg, xs)


def _head_body(xv_ref, w1_ref, b1_ref, w2_ref, b2_ref, o_ref):
    h = _leaky(jnp.dot(xv_ref[...], w1_ref[...],
                       preferred_element_type=jnp.float32) + b1_ref[...])
    o_ref[...] = jax.nn.sigmoid(
        jnp.dot(h, w2_ref[...], preferred_element_type=jnp.float32) + b2_ref[...])


def _head(xv, w1, b1, w2, b2):
    k = w1.shape[0]
    m = w2.shape[1]
    return pl.pallas_call(
        _head_body,
        in_specs=[
            pl.BlockSpec((1, k), lambda: (0, 0)),
            pl.BlockSpec((k, k), lambda: (0, 0)),
            pl.BlockSpec((1, k), lambda: (0, 0)),
            pl.BlockSpec((k, m), lambda: (0, 0)),
            pl.BlockSpec((1, m), lambda: (0, 0)),
        ],
        out_specs=pl.BlockSpec((1, m), lambda: (0, 0)),
        out_shape=jax.ShapeDtypeStruct((1, m), jnp.float32),
    )(xv, w1, b1.reshape(1, k), w2, b2.reshape(1, m))


def _make_segsum(n_pad, e_pad, cf, n_chunks, blk_k):
    cpc = n_chunks // _NC
    rows128 = e_pad // 128
    rt = rows128 // _NT
    gs = max(g for g in (40, 32, 24, 16, 8) if rt % g == 0)
    ngrp = rt // gs
    zr = n_pad // _NT

    mesh = plsc.VectorSubcoreMesh(core_axis_name="c", subcore_axis_name="s")

    def body(tbl_flat, src2d, dst2d, zbig, ones_h, sums_out, cnt_out,
             idx_sg, idx_dg, idx_z, rows_a, rows_b,
             sem, sem_a, sem_b, acc):
        c = lax.axis_index("c")
        s = lax.axis_index("s")
        it16 = lax.iota(jnp.int32, 16)

        bases = list(range(0, zr - 127, 128))
        if zr % 128:
            bases.append(zr - 128)

        def zero_rows():
            pltpu.sync_copy(zbig, rows_a.at[0])
            cps = []
            for t, b0 in enumerate(bases):
                base = s * zr + b0
                for l in range(8):
                    idx_z[t, pl.ds(l * 16, 16)] = base + l * 16 + it16
                cps.append(pltpu.async_copy(rows_a.at[0], acc.at[idx_z.at[t]],
                                            sem))
            for cp in cps:
                cp.wait()

        def copy_out(dst_hbm, row0):
            rds = [pltpu.async_copy(acc.at[pl.ds(s * zr + b0, 128)],
                                    (rows_a if t % 2 == 0 else rows_b).at[0],
                                    sem_a if t % 2 == 0 else sem_b)
                   for t, b0 in enumerate(bases[:1])]
            for t, b0 in enumerate(bases):
                if t + 1 < len(bases):
                    rds.append(pltpu.async_copy(
                        acc.at[pl.ds(s * zr + bases[t + 1], 128)],
                        (rows_a if (t + 1) % 2 == 0 else rows_b).at[0],
                        sem_a if (t + 1) % 2 == 0 else sem_b))
                rds[t].wait()
                pltpu.sync_copy((rows_a if t % 2 == 0 else rows_b).at[0],
                                dst_hbm.at[pl.ds(row0 + s * zr + b0, 128)])

        def drain(rows_i, sem_i):
            pltpu.make_async_copy(
                tbl_flat.at[pl.ds(0, 128)], rows_i.at[0], sem_i).wait()

        def fire(j, rows_i, sem_i):
            pltpu.async_copy(tbl_flat.at[idx_sg.at[j]], rows_i.at[0], sem_i)

        for k in range(cpc):
            g = c * cpc + k
            zero_rows()
            plsc.subcore_barrier()

            for grp in range(ngrp):
                base_row = s * rt + grp * gs
                pltpu.sync_copy(src2d.at[pl.ds(g * rows128 + base_row, gs)],
                                idx_sg.at[pl.ds(0, gs)])
                pltpu.sync_copy(dst2d.at[pl.ds(base_row, gs)],
                                idx_dg.at[pl.ds(0, gs)])
                fire(0, rows_a, sem_a)
                for j in range(0, gs, 2):
                    fire(j + 1, rows_b, sem_b)
                    drain(rows_a, sem_a)
                    pltpu.sync_copy(rows_a.at[0], acc.at[idx_dg.at[j]],
                                    add=True)
                    if j + 2 < gs:
                        fire(j + 2, rows_a, sem_a)
                    drain(rows_b, sem_b)
                    pltpu.sync_copy(rows_b.at[0], acc.at[idx_dg.at[j + 1]],
                                    add=True)
            plsc.subcore_barrier()

            copy_out(sums_out, g * n_pad)

        zero_rows()
        pltpu.sync_copy(ones_h, rows_a.at[0])
        plsc.subcore_barrier()
        for grp in range(ngrp):
            @pl.when(c == grp % _NC)
            def _():
                base_row = s * rt + grp * gs
                pltpu.sync_copy(dst2d.at[pl.ds(base_row, gs)],
                                idx_dg.at[pl.ds(0, gs)])
                for j in range(0, gs, 2):
                    cp1 = pltpu.async_copy(rows_a.at[0], acc.at[idx_dg.at[j]],
                                           sem_a, add=True)
                    cp2 = pltpu.async_copy(rows_a.at[0],
                                           acc.at[idx_dg.at[j + 1]],
                                           sem_b, add=True)
                    cp1.wait()
                    cp2.wait()
        plsc.subcore_barrier()
        copy_out(cnt_out, c * n_pad)

    return pl.kernel(
        body,
        out_type=(
            jax.ShapeDtypeStruct((n_chunks * n_pad, cf), jnp.float32),
            jax.ShapeDtypeStruct((2 * n_pad, cf), jnp.float32),
        ),
        mesh=mesh,
        scratch_types=[
            pltpu.VMEM((gs, 128), jnp.int32),
            pltpu.VMEM((gs, 128), jnp.int32),
            pltpu.VMEM((8, 128), jnp.int32),
            pltpu.VMEM((1, 128, cf), jnp.float32),
            pltpu.VMEM((1, 128, cf), jnp.float32),
            pltpu.SemaphoreType.DMA,
            pltpu.SemaphoreType.DMA,
            pltpu.SemaphoreType.DMA,
            pltpu.VMEM_SHARED((n_pad, cf), jnp.float32),
        ],
    )


def _segment_mean_inputs(table, src, dst, n_nodes, n_pad, cf, n_chunks, blk_k):
    e = src.shape[0]
    unit = 128 * 128
    e_pad = ((e + unit - 1) // unit) * unit
    npd = e_pad - e
    pad_src = (jnp.arange(npd, dtype=jnp.int32) * 7) % n_nodes
    pad_dst = n_nodes + (jnp.arange(npd, dtype=jnp.int32) % (n_pad - n_nodes))
    src_p = jnp.concatenate([src, pad_src])
    dst_p = jnp.concatenate([dst, pad_dst])
    offs = (jnp.arange(n_chunks, dtype=jnp.int32) * n_pad)[:, None]
    src2d = (src_p[None, :] + offs).reshape(n_chunks * e_pad // 128, 128)
    dst2d = dst_p.reshape(e_pad // 128, 128)
    src2d = jnp.concatenate([src2d, jnp.zeros((1, 128), jnp.int32)])
    dst2d = jnp.concatenate([dst2d, jnp.full((1, 128), n_nodes, jnp.int32)])
    tbl_p = jnp.concatenate(
        [table, jnp.zeros((n_pad - n_nodes, table.shape[1]), jnp.float32)])
    tbl_flat = tbl_p.reshape(n_pad, n_chunks, cf).transpose(1, 0, 2)
    tbl_flat = tbl_flat.reshape(n_chunks * n_pad, cf)
    zbig = jnp.zeros((128, cf), jnp.float32)
    ones_h = jnp.ones((128, cf), jnp.float32)
    fn = _make_segsum(n_pad, e_pad, cf, n_chunks, blk_k)
    sums_flat, cnt_flat = fn(tbl_flat, src2d, dst2d, zbig, ones_h)
    return sums_flat.reshape(n_chunks, n_pad, cf), cnt_flat.reshape(2, n_pad, cf)


def kernel(x, params, edge_index_diff, feats_size_list):
    p = params
    ei0, ei1 = edge_index_diff

    y = _mm_leaky(x, p['l0_w'], p['l0_b'], 1000)

    n0 = _F0 + _F1
    n0_pad = 9728
    sums0, cnt0 = _segment_mean_inputs(
        y[:n0], ei0[0], ei0[1], n0, n0_pad, cf=128, n_chunks=4, blk_k=1)
    g0 = _conv_post(sums0, cnt0, p['conv0_w'], p['conv0_b'],
                    p['ln0_g'], p['ln0_b'], 608)

    n1 = _F1 + _F2
    n1_pad = 2560
    xx1 = jnp.concatenate([g0[_F0:n0], y[n0:]], axis=0)
    sums1, cnt1 = _segment_mean_inputs(
        xx1, ei1[0] - _F0, ei1[1] - _F0, n1, n1_pad, cf=128, n_chunks=4,
        blk_k=1)
    g1 = _conv_post(sums1, cnt1, p['conv1_w'], p['conv1_b'],
                    p['ln1_g'], p['ln1_b'], 2560)

    x0p = jnp.concatenate([y[:_F0], jnp.zeros((7680 - _F0, _H), jnp.float32)])
    specs = [(x0p, _F0, 512), (g0[_F0:n0], _F1, 400), (g1[_F1:n1], _F2, 500)]
    at_ = []
    xv_list = []
    for i, (xi, n_valid, blk) in enumerate(specs):
        xs, lg = _att_a(xi, p['attl1_%d_w' % i], p['attl1_%d_b' % i],
                        p['att1_%d_w' % i], p['att1_%d_b' % i],
                        p['att2_%d_w' % i], p['att2_%d_b' % i],
                        p['att3_%d_w' % i], p['att3_%d_b' % i], blk)
        a, xv = _att_b(lg, xs, blk, n_valid)
        at_.append(a[:n_valid].T)
        xv_list.append(xv)

    xv = jnp.concatenate(xv_list, axis=-1)
    out = _head(xv, p['llast_w'], p['llast_b'], p['lcla_w'], p['lcla_b'])
    return out, at_
